# baseline (speedup 1.0000x reference)
import functools

import jax
import jax.numpy as jnp
from jax.experimental import pallas as pl
from jax.experimental.pallas import tpu as pltpu

N_EXPERTS = 16
INPUT_DIM = 1024
HIDDEN = 128
OUTPUT_DIM = 1024
B = 2048
K = 8


BT = 512


def _moe_kernel(x_ref, xb_ref, gw_ref, gb_ref, w1_ref, b1_ref, w2_ref,
                b2_ref, o_ref, wsel_ref):
    e = pl.program_id(1)

    @pl.when(e == 0)
    def _gate():
        x = x_ref[...]
        logits = jax.lax.dot_general(
            x, gw_ref[...], (((1,), (1,)), ((), ())),
            preferred_element_type=jnp.float32) + gb_ref[...]
        li = logits[:, :, None]
        lj = logits[:, None, :]
        idx = jax.lax.broadcasted_iota(jnp.int32, (1, N_EXPERTS, N_EXPERTS), 2)
        idx_t = jax.lax.broadcasted_iota(jnp.int32, (1, N_EXPERTS, N_EXPERTS), 1)
        beats = (lj > li) | ((lj == li) & (idx < idx_t))
        rank = jnp.sum(beats.astype(jnp.int32), axis=2)
        sel = rank < K
        neg = jnp.float32(-jnp.inf)
        masked = jnp.where(sel, logits, neg)
        m = jnp.max(masked, axis=1, keepdims=True)
        p = jnp.where(sel, jnp.exp(logits - m), 0.0)
        wsel_ref[...] = p / jnp.sum(p, axis=1, keepdims=True)

    cols = jax.lax.broadcasted_iota(jnp.int32, (1, N_EXPERTS), 1)
    w = jnp.sum(jnp.where(cols == e, wsel_ref[...], 0.0),
                axis=1, keepdims=True)
    h1 = jax.lax.dot_general(
        xb_ref[...], w1_ref[0], (((1,), (0,)), ((), ())),
        preferred_element_type=jnp.float32) + b1_ref[0]
    h1 = (jnp.maximum(h1, 0.0) * w).astype(jnp.bfloat16)
    contrib = jax.lax.dot_general(
        h1, w2_ref[0], (((1,), (0,)), ((), ())),
        preferred_element_type=jnp.float32) + w * b2_ref[0]

    @pl.when(e == 0)
    def _init():
        o_ref[...] = contrib

    @pl.when(e > 0)
    def _acc():
        o_ref[...] += contrib


@functools.partial(jax.jit, static_argnames=())
def _moe(x, gate_W, gate_b, expert_w1, expert_b1, expert_w2, expert_b2):
    gb = gate_b.reshape(1, N_EXPERTS)
    b1 = expert_b1.reshape(N_EXPERTS, 1, HIDDEN)
    b2 = expert_b2.reshape(N_EXPERTS, 1, OUTPUT_DIM)
    xb = x.astype(jnp.bfloat16)
    w1 = expert_w1.astype(jnp.bfloat16)
    w2 = expert_w2.astype(jnp.bfloat16)
    return pl.pallas_call(
        _moe_kernel,
        grid=(B // BT, N_EXPERTS),
        in_specs=[
            pl.BlockSpec((BT, INPUT_DIM), lambda i, e: (i, 0)),
            pl.BlockSpec((BT, INPUT_DIM), lambda i, e: (i, 0)),
            pl.BlockSpec((N_EXPERTS, INPUT_DIM), lambda i, e: (0, 0)),
            pl.BlockSpec((1, N_EXPERTS), lambda i, e: (0, 0)),
            pl.BlockSpec((1, INPUT_DIM, HIDDEN), lambda i, e: (e, 0, 0)),
            pl.BlockSpec((1, 1, HIDDEN), lambda i, e: (e, 0, 0)),
            pl.BlockSpec((1, HIDDEN, OUTPUT_DIM), lambda i, e: (e, 0, 0)),
            pl.BlockSpec((1, 1, OUTPUT_DIM), lambda i, e: (e, 0, 0)),
        ],
        out_specs=pl.BlockSpec((BT, OUTPUT_DIM), lambda i, e: (i, 0)),
        out_shape=jax.ShapeDtypeStruct((B, OUTPUT_DIM), jnp.float32),
        scratch_shapes=[pltpu.VMEM((BT, N_EXPERTS), jnp.float32)],
    )(x, xb, gate_W, gb, w1, b1, w2, b2)


def kernel(x, gate_W, gate_b, expert_w1, expert_b1, expert_w2, expert_b2, k):
    del k
    return _moe(x, gate_W, gate_b, expert_w1, expert_b1, expert_w2, expert_b2)


# speedup vs baseline: 1.1971x; 1.1971x over previous
import functools

import jax
import jax.numpy as jnp
from jax.experimental import pallas as pl
from jax.experimental.pallas import tpu as pltpu

N_EXPERTS = 16
INPUT_DIM = 1024
HIDDEN = 128
OUTPUT_DIM = 1024
B = 2048
K = 8


BT = 1024


def _moe_kernel(x_ref, gw_ref, gb_ref, w1_ref, b1_ref, w2_ref,
                b2_ref, o_ref, wsel_ref):
    e = pl.program_id(1)

    @pl.when(e == 0)
    def _gate():
        x = x_ref[...]
        logits = jax.lax.dot_general(
            x, gw_ref[...], (((1,), (1,)), ((), ())),
            preferred_element_type=jnp.float32) + gb_ref[...]
        li = logits[:, :, None]
        lj = logits[:, None, :]
        idx = jax.lax.broadcasted_iota(jnp.int32, (1, N_EXPERTS, N_EXPERTS), 2)
        idx_t = jax.lax.broadcasted_iota(jnp.int32, (1, N_EXPERTS, N_EXPERTS), 1)
        beats = (lj > li) | ((lj == li) & (idx < idx_t))
        rank = jnp.sum(beats.astype(jnp.int32), axis=2)
        sel = rank < K
        neg = jnp.float32(-jnp.inf)
        masked = jnp.where(sel, logits, neg)
        m = jnp.max(masked, axis=1, keepdims=True)
        p = jnp.where(sel, jnp.exp(logits - m), 0.0)
        wsel_ref[...] = p / jnp.sum(p, axis=1, keepdims=True)

    cols = jax.lax.broadcasted_iota(jnp.int32, (1, N_EXPERTS), 1)
    w = jnp.sum(jnp.where(cols == e, wsel_ref[...], 0.0),
                axis=1, keepdims=True)
    h1 = jax.lax.dot_general(
        x_ref[...], w1_ref[0], (((1,), (0,)), ((), ())),
        preferred_element_type=jnp.float32) + b1_ref[0]
    h1 = jnp.maximum(h1, 0.0) * w
    contrib = jax.lax.dot_general(
        h1, w2_ref[0], (((1,), (0,)), ((), ())),
        preferred_element_type=jnp.float32) + w * b2_ref[0]

    @pl.when(e == 0)
    def _init():
        o_ref[...] = contrib

    @pl.when(e > 0)
    def _acc():
        o_ref[...] += contrib


@functools.partial(jax.jit, static_argnames=())
def _moe(x, gate_W, gate_b, expert_w1, expert_b1, expert_w2, expert_b2):
    gb = gate_b.reshape(1, N_EXPERTS)
    b1 = expert_b1.reshape(N_EXPERTS, 1, HIDDEN)
    b2 = expert_b2.reshape(N_EXPERTS, 1, OUTPUT_DIM)
    return pl.pallas_call(
        _moe_kernel,
        grid=(B // BT, N_EXPERTS),
        in_specs=[
            pl.BlockSpec((BT, INPUT_DIM), lambda i, e: (i, 0)),
            pl.BlockSpec((N_EXPERTS, INPUT_DIM), lambda i, e: (0, 0)),
            pl.BlockSpec((1, N_EXPERTS), lambda i, e: (0, 0)),
            pl.BlockSpec((1, INPUT_DIM, HIDDEN), lambda i, e: (e, 0, 0)),
            pl.BlockSpec((1, 1, HIDDEN), lambda i, e: (e, 0, 0)),
            pl.BlockSpec((1, HIDDEN, OUTPUT_DIM), lambda i, e: (e, 0, 0)),
            pl.BlockSpec((1, 1, OUTPUT_DIM), lambda i, e: (e, 0, 0)),
        ],
        out_specs=pl.BlockSpec((BT, OUTPUT_DIM), lambda i, e: (i, 0)),
        out_shape=jax.ShapeDtypeStruct((B, OUTPUT_DIM), jnp.float32),
        scratch_shapes=[pltpu.VMEM((BT, N_EXPERTS), jnp.float32)],
    )(x, gate_W, gb, expert_w1, b1, expert_w2, b2)


def kernel(x, gate_W, gate_b, expert_w1, expert_b1, expert_w2, expert_b2, k):
    del k
    return _moe(x, gate_W, gate_b, expert_w1, expert_b1, expert_w2, expert_b2)


# speedup vs baseline: 2.1780x; 1.8194x over previous
import functools

import jax
import jax.numpy as jnp
from jax.experimental import pallas as pl
from jax.experimental.pallas import tpu as pltpu

N_EXPERTS = 16
INPUT_DIM = 1024
HIDDEN = 128
OUTPUT_DIM = 1024
B = 2048
K = 8
EH = N_EXPERTS * HIDDEN
BT = 1024


def _moe_kernel(x_ref, gw_ref, gb_ref, w1f_ref, b1f_ref, w2f_ref, b2m_ref,
                s_ref, o_ref):
    x = x_ref[...]
    logits = jax.lax.dot_general(
        x, gw_ref[...], (((1,), (1,)), ((), ())),
        preferred_element_type=jnp.float32) + gb_ref[...]
    li = logits[:, :, None]
    lj = logits[:, None, :]
    idx = jax.lax.broadcasted_iota(jnp.int32, (1, N_EXPERTS, N_EXPERTS), 2)
    idx_t = jax.lax.broadcasted_iota(jnp.int32, (1, N_EXPERTS, N_EXPERTS), 1)
    beats = (lj > li) | ((lj == li) & (idx < idx_t))
    rank = jnp.sum(beats.astype(jnp.int32), axis=2)
    sel = rank < K
    masked = jnp.where(sel, logits, -jnp.inf)
    m = jnp.max(masked, axis=1, keepdims=True)
    p = jnp.where(sel, jnp.exp(logits - m), 0.0)
    wsel = p / jnp.sum(p, axis=1, keepdims=True)

    h = jax.lax.dot_general(
        x, w1f_ref[...], (((1,), (0,)), ((), ())),
        preferred_element_type=jnp.float32) + b1f_ref[...]
    scale = jax.lax.dot_general(
        wsel, s_ref[...], (((1,), (0,)), ((), ())),
        preferred_element_type=jnp.float32)
    g = jnp.maximum(h, 0.0) * scale
    out = jax.lax.dot_general(
        g, w2f_ref[...], (((1,), (0,)), ((), ())),
        preferred_element_type=jnp.float32)
    out += jax.lax.dot_general(
        wsel, b2m_ref[...], (((1,), (0,)), ((), ())),
        preferred_element_type=jnp.float32)
    o_ref[...] = out


@jax.jit
def _moe(x, gate_W, gate_b, expert_w1, expert_b1, expert_w2, expert_b2):
    gb = gate_b.reshape(1, N_EXPERTS)
    w1f = expert_w1.transpose(1, 0, 2).reshape(INPUT_DIM, EH)
    b1f = expert_b1.reshape(1, EH)
    w2f = expert_w2.reshape(EH, OUTPUT_DIM)
    cols = jnp.arange(EH, dtype=jnp.int32) // HIDDEN
    s = (cols[None, :] == jnp.arange(N_EXPERTS, dtype=jnp.int32)[:, None]
         ).astype(jnp.float32)
    return pl.pallas_call(
        _moe_kernel,
        grid=(B // BT,),
        in_specs=[
            pl.BlockSpec((BT, INPUT_DIM), lambda i: (i, 0)),
            pl.BlockSpec((N_EXPERTS, INPUT_DIM), lambda i: (0, 0)),
            pl.BlockSpec((1, N_EXPERTS), lambda i: (0, 0)),
            pl.BlockSpec((INPUT_DIM, EH), lambda i: (0, 0)),
            pl.BlockSpec((1, EH), lambda i: (0, 0)),
            pl.BlockSpec((EH, OUTPUT_DIM), lambda i: (0, 0)),
            pl.BlockSpec((N_EXPERTS, OUTPUT_DIM), lambda i: (0, 0)),
            pl.BlockSpec((N_EXPERTS, EH), lambda i: (0, 0)),
        ],
        out_specs=pl.BlockSpec((BT, OUTPUT_DIM), lambda i: (i, 0)),
        out_shape=jax.ShapeDtypeStruct((B, OUTPUT_DIM), jnp.float32),
    )(x, gate_W, gb, w1f, b1f, w2f, expert_b2, s)


def kernel(x, gate_W, gate_b, expert_w1, expert_b1, expert_w2, expert_b2, k):
    del k
    return _moe(x, gate_W, gate_b, expert_w1, expert_b1, expert_w2, expert_b2)
